# baseline (speedup 1.0000x reference)
import functools

import jax
import jax.numpy as jnp
from jax import lax
from jax.experimental import pallas as pl
from jax.experimental.pallas import tpu as pltpu
from jax.experimental.pallas import tpu_sc as plsc

_VOX = 96 * 96 * 96
_NC, _NS, _L = 2, 16, 16
_NW = _NC * _NS
_PER_W = (2 * _VOX) // _NW
_CHUNK = 6912
_NCHUNK = _PER_W // _CHUNK


def _sc_partials_body(pred_hbm, tgt_hbm, out_hbm,
                      pb0, pb1, pb2, pb3, tb1, tb2, tb3, obuf):
    wid = lax.axis_index("s") * _NC + lax.axis_index("c")
    b = wid // _NS
    wbase = (wid % _NS) * _PER_W
    pbufs = (pb0, pb1, pb2, pb3)
    tbufs = (tb1, tb2, tb3)

    def chunk_body(k, accs):
        base = wbase + k * _CHUNK
        for c in range(4):
            pltpu.sync_copy(pred_hbm.at[b * 4 + c, pl.ds(base, _CHUNK)],
                            pbufs[c])
        for c in range(3):
            pltpu.sync_copy(tgt_hbm.at[b * 4 + 1 + c, pl.ds(base, _CHUNK)],
                            tbufs[c])

        def body(i, accs):
            (ai1, ai2, ai3, ap1, ap2, ap3, at1, at2, at3) = accs
            s = pl.ds(i * _L, _L)
            p0 = pb0[s]
            p1 = pb1[s]
            p2 = pb2[s]
            p3 = pb3[s]
            t1 = tb1[s]
            t2 = tb2[s]
            t3 = tb3[s]
            is1 = (p1 > p0) & (p1 >= p2) & (p1 >= p3)
            is2 = (p2 > p0) & (p2 > p1) & (p2 >= p3)
            is3 = (p3 > p0) & (p3 > p1) & (p3 > p2)
            zero = jnp.zeros((_L,), jnp.float32)
            one = jnp.ones((_L,), jnp.float32)
            return (
                ai1 + jnp.where(is1, t1, zero),
                ai2 + jnp.where(is2, t2, zero),
                ai3 + jnp.where(is3, t3, zero),
                ap1 + jnp.where(is1, one, zero),
                ap2 + jnp.where(is2, one, zero),
                ap3 + jnp.where(is3, one, zero),
                at1 + t1,
                at2 + t2,
                at3 + t3,
            )

        return lax.fori_loop(0, _CHUNK // _L, body, accs)

    zeros = tuple(jnp.zeros((_L,), jnp.float32) for _ in range(9))
    accs = lax.fori_loop(0, _NCHUNK, chunk_body, zeros)
    for k in range(9):
        obuf[k, :] = accs[k]
    pltpu.sync_copy(obuf, out_hbm.at[wid])


@functools.cache
def _sc_partials():
    return pl.kernel(
        _sc_partials_body,
        mesh=plsc.VectorSubcoreMesh(core_axis_name="c", subcore_axis_name="s"),
        out_type=jax.ShapeDtypeStruct((_NW, 9, _L), jnp.float32),
        scratch_types=[pltpu.VMEM((_CHUNK,), jnp.float32)] * 7
        + [pltpu.VMEM((9, _L), jnp.float32)],
    )


def _finalize_body(part_ref, pct_ref, tcl_ref, out_ref):
    p = part_ref[...]
    s = [jnp.sum(p[:, k, :]) for k in range(9)]

    mdsc = jnp.float32(0.0)
    mppv = jnp.float32(0.0)
    msen = jnp.float32(0.0)
    cnt_t = jnp.float32(0.0)
    cnt_p = jnp.float32(0.0)
    for c in range(3):
        inter, psum, tsum = s[c], s[3 + c], s[6 + c]
        valid_t = (tsum > 0).astype(jnp.float32)
        valid_p = (psum > 0).astype(jnp.float32)
        cnt_t = cnt_t + valid_t
        cnt_p = cnt_p + valid_p
        mdsc = mdsc + valid_t * (2.0 * inter + 1e-5) / (psum + tsum + 1e-5)
        mppv = mppv + valid_p * (inter + 1.0) / (psum + 1.0)
        msen = msen + valid_t * (inter + 1.0) / (tsum + 1.0)
    dsc = jnp.where(cnt_t > 0, mdsc / jnp.maximum(cnt_t, 1.0), mdsc)
    ppv = jnp.where(cnt_p > 0, mppv / jnp.maximum(cnt_p, 1.0), mppv)
    sen = jnp.where(cnt_t > 0, msen / jnp.maximum(cnt_t, 1.0), msen)

    pct = pct_ref[...]
    best = pct[0:1, :]
    idx = jnp.zeros((1, 32), jnp.int32)
    for k in range(1, 4):
        row = pct[k:k + 1, :]
        take = row > best
        idx = jnp.where(take, jnp.int32(k), idx)
        best = jnp.where(take, row, best)
    eq = (idx == tcl_ref[...]).astype(jnp.float32)
    ii = lax.broadcasted_iota(jnp.int32, (1, 32), 1)
    pred_actual = jnp.sum(jnp.where(ii < 2, eq, 0.0)) / 2.0
    pred_stat = jnp.sum(jnp.where(ii >= 2, eq, 0.0)) / 30.0

    oi = lax.broadcasted_iota(jnp.int32, (1, 8), 1)
    out = (jnp.where(oi == 0, dsc, 0.0)
           + jnp.where(oi == 1, ppv, 0.0)
           + jnp.where(oi == 2, sen, 0.0)
           + jnp.where(oi == 3, pred_actual, 0.0)
           + jnp.where(oi == 4, pred_stat, 0.0))
    out_ref[...] = out


def kernel(pred_mask, pred_classes, target_mask, target_class):
    pred_flat = pred_mask.reshape(8, _VOX)
    tgt_flat = target_mask.reshape(8, _VOX)
    partials = _sc_partials()(pred_flat, tgt_flat)
    pct = pred_classes.T
    tcl = target_class.astype(jnp.int32).reshape(1, 32)
    out = pl.pallas_call(
        _finalize_body,
        out_shape=jax.ShapeDtypeStruct((1, 8), jnp.float32),
    )(partials, pct, tcl)
    return out[0, :5]


# speedup vs baseline: 9.0300x; 9.0300x over previous
import functools

import jax
import jax.numpy as jnp
from jax import lax
from jax.experimental import pallas as pl
from jax.experimental.pallas import tpu as pltpu
from jax.experimental.pallas import tpu_sc as plsc

_VOX = 96 * 96 * 96
_ROW = 96
_NROW = _VOX // _ROW
_NC, _NS, _L = 2, 16, 16
_NW = _NC * _NS
_ROW_W = _NROW // _NS
_CROW = 72
_NCHUNK = _ROW_W // _CROW
_LB = _ROW // _L


def _sc_partials_body(pred_hbm, tgt_hbm, out_hbm,
                      pb0, pb1, pb2, pb3, tb1, tb2, tb3, obuf):
    wid = lax.axis_index("s") * _NC + lax.axis_index("c")
    b = wid // _NS
    wbase = (wid % _NS) * _ROW_W
    pbufs = (pb0, pb1, pb2, pb3)
    tbufs = (tb1, tb2, tb3)

    def chunk_body(k, accs):
        base = wbase + k * _CROW
        for c in range(4):
            pltpu.sync_copy(pred_hbm.at[b * 4 + c, pl.ds(base, _CROW), :],
                            pbufs[c])
        for c in range(3):
            pltpu.sync_copy(tgt_hbm.at[b * 4 + 1 + c, pl.ds(base, _CROW), :],
                            tbufs[c])

        def body(r, accs):
            for l in range(_LB):
                (ai1, ai2, ai3, ap1, ap2, ap3, at1, at2, at3) = accs
                s = pl.ds(l * _L, _L)
                p0 = pb0[r, s]
                p1 = pb1[r, s]
                p2 = pb2[r, s]
                p3 = pb3[r, s]
                t1 = tb1[r, s]
                t2 = tb2[r, s]
                t3 = tb3[r, s]
                is1 = (p1 > p0) & (p1 >= p2) & (p1 >= p3)
                is2 = (p2 > p0) & (p2 > p1) & (p2 >= p3)
                is3 = (p3 > p0) & (p3 > p1) & (p3 > p2)
                zero = jnp.zeros((_L,), jnp.float32)
                one = jnp.ones((_L,), jnp.float32)
                accs = (
                    ai1 + jnp.where(is1, t1, zero),
                    ai2 + jnp.where(is2, t2, zero),
                    ai3 + jnp.where(is3, t3, zero),
                    ap1 + jnp.where(is1, one, zero),
                    ap2 + jnp.where(is2, one, zero),
                    ap3 + jnp.where(is3, one, zero),
                    at1 + t1,
                    at2 + t2,
                    at3 + t3,
                )
            return accs

        return lax.fori_loop(0, _CROW, body, accs)

    zeros = tuple(jnp.zeros((_L,), jnp.float32) for _ in range(9))
    accs = lax.fori_loop(0, _NCHUNK, chunk_body, zeros)
    for k in range(9):
        obuf[k, :] = accs[k]
    pltpu.sync_copy(obuf, out_hbm.at[wid])


@functools.cache
def _sc_partials():
    return pl.kernel(
        _sc_partials_body,
        mesh=plsc.VectorSubcoreMesh(core_axis_name="c", subcore_axis_name="s"),
        out_type=jax.ShapeDtypeStruct((_NW, 9, _L), jnp.float32),
        scratch_types=[pltpu.VMEM((_CROW, _ROW), jnp.float32)] * 7
        + [pltpu.VMEM((9, _L), jnp.float32)],
    )


def _finalize_body(part_ref, pct_ref, tcl_ref, out_ref):
    p = part_ref[...]
    s = [jnp.sum(p[:, k, :]) for k in range(9)]

    mdsc = jnp.float32(0.0)
    mppv = jnp.float32(0.0)
    msen = jnp.float32(0.0)
    cnt_t = jnp.float32(0.0)
    cnt_p = jnp.float32(0.0)
    for c in range(3):
        inter, psum, tsum = s[c], s[3 + c], s[6 + c]
        valid_t = (tsum > 0).astype(jnp.float32)
        valid_p = (psum > 0).astype(jnp.float32)
        cnt_t = cnt_t + valid_t
        cnt_p = cnt_p + valid_p
        mdsc = mdsc + valid_t * (2.0 * inter + 1e-5) / (psum + tsum + 1e-5)
        mppv = mppv + valid_p * (inter + 1.0) / (psum + 1.0)
        msen = msen + valid_t * (inter + 1.0) / (tsum + 1.0)
    dsc = jnp.where(cnt_t > 0, mdsc / jnp.maximum(cnt_t, 1.0), mdsc)
    ppv = jnp.where(cnt_p > 0, mppv / jnp.maximum(cnt_p, 1.0), mppv)
    sen = jnp.where(cnt_t > 0, msen / jnp.maximum(cnt_t, 1.0), msen)

    pct = pct_ref[...]
    best = pct[0:1, :]
    idx = jnp.zeros((1, 32), jnp.int32)
    for k in range(1, 4):
        row = pct[k:k + 1, :]
        take = row > best
        idx = jnp.where(take, jnp.int32(k), idx)
        best = jnp.where(take, row, best)
    eq = (idx == tcl_ref[...]).astype(jnp.float32)
    ii = lax.broadcasted_iota(jnp.int32, (1, 32), 1)
    pred_actual = jnp.sum(jnp.where(ii < 2, eq, 0.0)) / 2.0
    pred_stat = jnp.sum(jnp.where(ii >= 2, eq, 0.0)) / 30.0

    oi = lax.broadcasted_iota(jnp.int32, (1, 8), 1)
    out = (jnp.where(oi == 0, dsc, 0.0)
           + jnp.where(oi == 1, ppv, 0.0)
           + jnp.where(oi == 2, sen, 0.0)
           + jnp.where(oi == 3, pred_actual, 0.0)
           + jnp.where(oi == 4, pred_stat, 0.0))
    out_ref[...] = out


def kernel(pred_mask, pred_classes, target_mask, target_class):
    pred_flat = pred_mask.reshape(8, _NROW, _ROW)
    tgt_flat = target_mask.reshape(8, _NROW, _ROW)
    partials = _sc_partials()(pred_flat, tgt_flat)
    pct = pred_classes.T
    tcl = target_class.astype(jnp.int32).reshape(1, 32)
    out = pl.pallas_call(
        _finalize_body,
        out_shape=jax.ShapeDtypeStruct((1, 8), jnp.float32),
    )(partials, pct, tcl)
    return out[0, :5]


# speedup vs baseline: 11.2748x; 1.2486x over previous
import functools

import jax
import jax.numpy as jnp
from jax import lax
from jax.experimental import pallas as pl
from jax.experimental.pallas import tpu as pltpu
from jax.experimental.pallas import tpu_sc as plsc

_VOX = 96 * 96 * 96
_ROW = 96
_NROW = _VOX // _ROW
_NC, _NS, _L = 2, 16, 16
_NW = _NC * _NS
_ROW_W = _NROW // _NS
_CROW = 64
_NCHUNK = _ROW_W // _CROW
_LB = _ROW // _L


def _sc_partials_body(pred_hbm, tgt_hbm, out_hbm, *refs):
    bufs = refs[:14]
    obuf = refs[14]
    sems = refs[15:17]
    wid = lax.axis_index("s") * _NC + lax.axis_index("c")
    b = wid // _NS
    wbase = (wid % _NS) * _ROW_W

    def copies(k, ring):
        base = wbase + k * _CROW
        bs = bufs[7 * ring:7 * ring + 7]
        descs = []
        for c in range(4):
            descs.append(pltpu.make_async_copy(
                pred_hbm.at[b * 4 + c, pl.ds(base, _CROW), :],
                bs[c], sems[ring]))
        for c in range(3):
            descs.append(pltpu.make_async_copy(
                tgt_hbm.at[b * 4 + 1 + c, pl.ds(base, _CROW), :],
                bs[4 + c], sems[ring]))
        return descs

    def compute(ring, accs):
        pb0, pb1, pb2, pb3, tb1, tb2, tb3 = bufs[7 * ring:7 * ring + 7]

        def body(r, accs):
            for l in range(_LB):
                (ai1, ai2, ai3, ap1, ap2, ap3, at1, at2, at3) = accs
                s = pl.ds(l * _L, _L)
                p0 = pb0[r, s]
                p1 = pb1[r, s]
                p2 = pb2[r, s]
                p3 = pb3[r, s]
                t1 = tb1[r, s]
                t2 = tb2[r, s]
                t3 = tb3[r, s]
                is1 = (p1 > p0) & (p1 >= p2) & (p1 >= p3)
                is2 = (p2 > p0) & (p2 > p1) & (p2 >= p3)
                is3 = (p3 > p0) & (p3 > p1) & (p3 > p2)
                zero = jnp.zeros((_L,), jnp.float32)
                one = jnp.ones((_L,), jnp.float32)
                accs = (
                    ai1 + jnp.where(is1, t1, zero),
                    ai2 + jnp.where(is2, t2, zero),
                    ai3 + jnp.where(is3, t3, zero),
                    ap1 + jnp.where(is1, one, zero),
                    ap2 + jnp.where(is2, one, zero),
                    ap3 + jnp.where(is3, one, zero),
                    at1 + t1,
                    at2 + t2,
                    at3 + t3,
                )
            return accs

        return lax.fori_loop(0, _CROW, body, accs)

    accs = tuple(jnp.zeros((_L,), jnp.float32) for _ in range(9))
    for d in copies(0, 0):
        d.start()
    for k in range(_NCHUNK):
        ring = k % 2
        if k + 1 < _NCHUNK:
            for d in copies(k + 1, 1 - ring):
                d.start()
        for d in copies(k, ring):
            d.wait()
        accs = compute(ring, accs)
    for k in range(9):
        obuf[k, :] = accs[k]
    pltpu.sync_copy(obuf, out_hbm.at[wid])


@functools.cache
def _sc_partials():
    return pl.kernel(
        _sc_partials_body,
        mesh=plsc.VectorSubcoreMesh(core_axis_name="c", subcore_axis_name="s"),
        out_type=jax.ShapeDtypeStruct((_NW, 9, _L), jnp.float32),
        scratch_types=[pltpu.VMEM((_CROW, _ROW), jnp.float32)] * 14
        + [pltpu.VMEM((9, _L), jnp.float32)]
        + [pltpu.SemaphoreType.DMA] * 2,
    )


def _finalize_body(part_ref, pct_ref, tcl_ref, out_ref):
    p = part_ref[...]
    s = [jnp.sum(p[:, k, :]) for k in range(9)]

    mdsc = jnp.float32(0.0)
    mppv = jnp.float32(0.0)
    msen = jnp.float32(0.0)
    cnt_t = jnp.float32(0.0)
    cnt_p = jnp.float32(0.0)
    for c in range(3):
        inter, psum, tsum = s[c], s[3 + c], s[6 + c]
        valid_t = (tsum > 0).astype(jnp.float32)
        valid_p = (psum > 0).astype(jnp.float32)
        cnt_t = cnt_t + valid_t
        cnt_p = cnt_p + valid_p
        mdsc = mdsc + valid_t * (2.0 * inter + 1e-5) / (psum + tsum + 1e-5)
        mppv = mppv + valid_p * (inter + 1.0) / (psum + 1.0)
        msen = msen + valid_t * (inter + 1.0) / (tsum + 1.0)
    dsc = jnp.where(cnt_t > 0, mdsc / jnp.maximum(cnt_t, 1.0), mdsc)
    ppv = jnp.where(cnt_p > 0, mppv / jnp.maximum(cnt_p, 1.0), mppv)
    sen = jnp.where(cnt_t > 0, msen / jnp.maximum(cnt_t, 1.0), msen)

    pct = pct_ref[...]
    best = pct[0:1, :]
    idx = jnp.zeros((1, 32), jnp.int32)
    for k in range(1, 4):
        row = pct[k:k + 1, :]
        take = row > best
        idx = jnp.where(take, jnp.int32(k), idx)
        best = jnp.where(take, row, best)
    eq = (idx == tcl_ref[...]).astype(jnp.float32)
    ii = lax.broadcasted_iota(jnp.int32, (1, 32), 1)
    pred_actual = jnp.sum(jnp.where(ii < 2, eq, 0.0)) / 2.0
    pred_stat = jnp.sum(jnp.where(ii >= 2, eq, 0.0)) / 30.0

    oi = lax.broadcasted_iota(jnp.int32, (1, 8), 1)
    out = (jnp.where(oi == 0, dsc, 0.0)
           + jnp.where(oi == 1, ppv, 0.0)
           + jnp.where(oi == 2, sen, 0.0)
           + jnp.where(oi == 3, pred_actual, 0.0)
           + jnp.where(oi == 4, pred_stat, 0.0))
    out_ref[...] = out


def kernel(pred_mask, pred_classes, target_mask, target_class):
    pred_flat = pred_mask.reshape(8, _NROW, _ROW)
    tgt_flat = target_mask.reshape(8, _NROW, _ROW)
    partials = _sc_partials()(pred_flat, tgt_flat)
    pct = pred_classes.T
    tcl = target_class.astype(jnp.int32).reshape(1, 32)
    out = pl.pallas_call(
        _finalize_body,
        out_shape=jax.ShapeDtypeStruct((1, 8), jnp.float32),
    )(partials, pct, tcl)
    return out[0, :5]
